# baseline (speedup 1.0000x reference)
import functools

import jax
import jax.numpy as jnp
from jax import lax
from jax.experimental import pallas as pl
from jax.experimental.pallas import tpu as pltpu
from jax.experimental.pallas import tpu_sc as plsc

T = 2048
D = 1024
F = 2048
E = 8
K = 2
S = T * K

BT = 256
NB = S // BT + E
NP = NB * BT

NC = 2
NS = 16
NW = NC * NS
RPW = NP // NW
TPW = T // NW


_RTB = 512


def _router_kernel(x_ref, wr_ref, eid_ref, w_ref):
    xb = x_ref[...]
    logits = jnp.dot(xb, wr_ref[...], preferred_element_type=jnp.float32)
    p = jax.nn.softmax(logits, axis=-1)
    ii = lax.broadcasted_iota(jnp.int32, p.shape, 1)
    m1 = jnp.max(p, axis=-1, keepdims=True)
    i1 = jnp.min(jnp.where(p >= m1, ii, E), axis=-1, keepdims=True)
    p2 = jnp.where(ii == i1, -jnp.inf, p)
    m2 = jnp.max(p2, axis=-1, keepdims=True)
    i2 = jnp.min(jnp.where(p2 >= m2, ii, E), axis=-1, keepdims=True)
    eid_ref[...] = jnp.concatenate([i1, i2], axis=1)
    w_ref[...] = jnp.concatenate([m1, m2], axis=1)


def _router(x, Wr):
    return pl.pallas_call(
        _router_kernel,
        grid=(T // _RTB,),
        in_specs=[
            pl.BlockSpec((_RTB, D), lambda t: (t, 0)),
            pl.BlockSpec((D, E), lambda t: (0, 0)),
        ],
        out_specs=[
            pl.BlockSpec((_RTB, K), lambda t: (t, 0)),
            pl.BlockSpec((_RTB, K), lambda t: (t, 0)),
        ],
        out_shape=[
            jax.ShapeDtypeStruct((T, K), jnp.int32),
            jax.ShapeDtypeStruct((T, K), jnp.float32),
        ],
    )(x, Wr)


def _lane(vec, i):
    lanes = lax.iota(jnp.int32, 16)
    return jnp.sum(jnp.where(lanes == i, vec, 0), axis=0)


def _route_gather_kernel(eidf_hbm, wf_hbm, x_hbm,
                         destp_hbm, tok_hbm, ws_hbm, be_hbm, nu_hbm, xs_hbm,
                         eid_v, wf_v, rank_v, destp_v, tok_v, ws_v,
                         be_v, nu_v, sh_tok, idx_v, rows_v, sem, sem2):
    cid = lax.axis_index("c")
    sid = lax.axis_index("s")
    lanes = lax.iota(jnp.int32, 16)

    @pl.when(sid == 0)
    def _routing():
        pltpu.sync_copy(eidf_hbm, eid_v)
        pltpu.sync_copy(wf_hbm, wf_v)

        def p1(i, c):
            v = eid_v[pl.ds(i * 16, 16)]
            seg = jnp.zeros((16,), jnp.int32)
            upd = jnp.zeros((16,), jnp.int32)
            for e in range(E):
                m = v == e
                pc = plsc.cumsum(jnp.where(m, 1, 0))
                seg = jnp.where(m, _lane(c, e) + pc - 1, seg)
                upd = jnp.where(lanes == e, _lane(pc, 15), upd)
            rank_v[pl.ds(i * 16, 16)] = seg
            return c + upd

        cnt = lax.fori_loop(0, S // 16, p1, jnp.zeros((16,), jnp.int32))

        pcnt = ((cnt + (BT - 1)) // BT) * BT
        csum = plsc.cumsum(pcnt)
        padoff = csum - pcnt
        nu_slots = _lane(csum, E - 1)
        nu_blocks = nu_slots // BT
        last_e = jnp.max(jnp.where(pcnt > 0, lanes, -1), axis=0)
        for r in range(NB // 16 + 1):
            bio = lanes + r * 16
            boff = bio * BT
            bev = jnp.zeros((16,), jnp.int32)
            for e in range(E):
                bev = bev + jnp.where(boff >= _lane(padoff, e), 1, 0)
            bev = bev - 1
            bev = jnp.where(bio >= nu_blocks, last_e, bev)
            be_v[pl.ds(r * 16, 16)] = bev
        nu_v[...] = jnp.broadcast_to(nu_blocks, (16,))

        def pz(i, _):
            z16 = i * 16
            tok_v[pl.ds(z16, 16)] = jnp.zeros((16,), jnp.int32)
            ws_v[pl.ds(z16, 16)] = jnp.zeros((16,), jnp.float32)
            return 0

        lax.fori_loop(0, NP // 16, pz, 0)

        def p3(i, _):
            v = eid_v[pl.ds(i * 16, 16)]
            r = rank_v[pl.ds(i * 16, 16)]
            w = wf_v[pl.ds(i * 16, 16)]
            po = jnp.zeros((16,), jnp.int32)
            for e in range(E):
                po = jnp.where(v == e, _lane(padoff, e), po)
            dest = po + r
            s_vec = lanes + i * 16
            loc = (s_vec & 1) * T + (s_vec >> 1)
            plsc.store_scatter(destp_v, [loc], dest)
            plsc.store_scatter(tok_v, [dest], s_vec >> 1)
            plsc.store_scatter(ws_v, [dest], w)
            return 0

        lax.fori_loop(0, S // 16, p3, 0)

        @pl.when(cid == 0)
        def _publish_hbm():
            pltpu.sync_copy(destp_v, destp_hbm)
            pltpu.sync_copy(tok_v, tok_hbm)
            pltpu.sync_copy(ws_v, ws_hbm)
            pltpu.sync_copy(be_v, be_hbm)
            pltpu.sync_copy(nu_v, nu_hbm)

        pltpu.sync_copy(tok_v, sh_tok)

    plsc.subcore_barrier()

    wid = cid * NS + sid
    base = wid * RPW
    pltpu.sync_copy(sh_tok.at[pl.ds(base, RPW)], idx_v)
    for j in range(RPW // 64):
        pltpu.async_copy(x_hbm.at[idx_v.at[pl.ds(j * 64, 64)]],
                         rows_v, sem).wait()
        pltpu.sync_copy(rows_v, xs_hbm.at[pl.ds(base + j * 64, 64)])


def _route_gather(eidf, wf, x):
    mesh = plsc.VectorSubcoreMesh(core_axis_name="c", subcore_axis_name="s")
    f = pl.kernel(
        _route_gather_kernel,
        out_type=[
            jax.ShapeDtypeStruct((K * T,), jnp.int32),
            jax.ShapeDtypeStruct((NP,), jnp.int32),
            jax.ShapeDtypeStruct((NP,), jnp.float32),
            jax.ShapeDtypeStruct((32,), jnp.int32),
            jax.ShapeDtypeStruct((16,), jnp.int32),
            jax.ShapeDtypeStruct((NP, D), jnp.float32),
        ],
        mesh=mesh,
        scratch_types=[
            pltpu.VMEM((S,), jnp.int32),
            pltpu.VMEM((S,), jnp.float32),
            pltpu.VMEM((S,), jnp.int32),
            pltpu.VMEM((S,), jnp.int32),
            pltpu.VMEM((NP,), jnp.int32),
            pltpu.VMEM((NP,), jnp.float32),
            pltpu.VMEM((32,), jnp.int32),
            pltpu.VMEM((16,), jnp.int32),
            pltpu.VMEM_SHARED((NP,), jnp.int32),
            pltpu.VMEM((RPW,), jnp.int32),
            pltpu.VMEM((64, D), jnp.float32),
            pltpu.SemaphoreType.DMA,
            pltpu.SemaphoreType.DMA,
        ],
        compiler_params=pltpu.CompilerParams(needs_layout_passes=False),
    )
    return f(eidf, wf, x)


def _ffn_kernel(be_ref, nu_ref, xs_ref, w1_ref, b1_ref, w2_ref, b2_ref,
                ws_ref, ys_ref):
    b = pl.program_id(0)

    @pl.when(b < nu_ref[0])
    def _():
        xb = xs_ref[...]
        h = jnp.dot(xb, w1_ref[0], preferred_element_type=jnp.float32)
        h = jax.nn.gelu(h + b1_ref[0])
        y = jnp.dot(h, w2_ref[0], preferred_element_type=jnp.float32)
        y = y + b2_ref[0]
        ys_ref[...] = y * ws_ref[0]


def _ffn(be, nu, xs, W1, b1r, W2, b2r, ws3):
    grid_spec = pltpu.PrefetchScalarGridSpec(
        num_scalar_prefetch=2,
        grid=(NB,),
        in_specs=[
            pl.BlockSpec((BT, D),
                         lambda b, be, nu: (jnp.minimum(b, nu[0] - 1), 0)),
            pl.BlockSpec((1, D, F), lambda b, be, nu: (be[b], 0, 0)),
            pl.BlockSpec((1, 1, F), lambda b, be, nu: (be[b], 0, 0)),
            pl.BlockSpec((1, F, D), lambda b, be, nu: (be[b], 0, 0)),
            pl.BlockSpec((1, 1, D), lambda b, be, nu: (be[b], 0, 0)),
            pl.BlockSpec((1, BT, 1), lambda b, be, nu: (b, 0, 0)),
        ],
        out_specs=pl.BlockSpec((BT, D), lambda b, be, nu: (b, 0)),
    )
    return pl.pallas_call(
        _ffn_kernel,
        grid_spec=grid_spec,
        out_shape=jax.ShapeDtypeStruct((NP, D), jnp.float32),
        compiler_params=pltpu.CompilerParams(
            dimension_semantics=("arbitrary",),
        ),
    )(be, nu, xs, W1, b1r, W2, b2r, ws3)


def _combine_kernel(ysw_hbm, destp_hbm, out_hbm, i0, i1, a_v, b_v, sem):
    cid = lax.axis_index("c")
    sid = lax.axis_index("s")
    wid = cid * NS + sid
    t0 = wid * TPW
    for ch in range(TPW // 16):
        tb = t0 + ch * 16
        pltpu.sync_copy(destp_hbm.at[pl.ds(tb, 16)], i0)
        pltpu.sync_copy(destp_hbm.at[pl.ds(T + tb, 16)], i1)
        pltpu.async_copy(ysw_hbm.at[i0], a_v, sem).wait()
        pltpu.async_copy(ysw_hbm.at[i1], b_v, sem).wait()

        def add_row(j, _):
            for c in range(D // 16):
                sl = pl.ds(c * 16, 16)
                a_v[j, sl] = a_v[j, sl] + b_v[j, sl]
            return 0

        lax.fori_loop(0, 16, add_row, 0)
        pltpu.sync_copy(a_v, out_hbm.at[pl.ds(tb, 16)])


def _combine(ysw, destp):
    mesh = plsc.VectorSubcoreMesh(core_axis_name="c", subcore_axis_name="s")
    f = pl.kernel(
        _combine_kernel,
        out_type=jax.ShapeDtypeStruct((T, D), jnp.float32),
        mesh=mesh,
        scratch_types=[
            pltpu.VMEM((16,), jnp.int32),
            pltpu.VMEM((16,), jnp.int32),
            pltpu.VMEM((16, D), jnp.float32),
            pltpu.VMEM((16, D), jnp.float32),
            pltpu.SemaphoreType.DMA,
        ],
        compiler_params=pltpu.CompilerParams(needs_layout_passes=False),
    )
    return f(ysw, destp)


def kernel(x, Wr, W1, b1, W2, b2):
    eid2, w2 = _router(x, Wr)
    eidf = eid2.reshape(S)
    wf = w2.reshape(S)
    destp, tok, ws, be, nu, xs = _route_gather(eidf, wf, x)
    b1r = b1.reshape(E, 1, F)
    b2r = b2.reshape(E, 1, D)
    ws3 = ws.reshape(NB, BT, 1)
    ysw = _ffn(be, nu, xs, W1, b1r, W2, b2r, ws3)
    return _combine(ysw, destp)


# speedup vs baseline: 1.0238x; 1.0238x over previous
import functools

import jax
import jax.numpy as jnp
from jax import lax
from jax.experimental import pallas as pl
from jax.experimental.pallas import tpu as pltpu
from jax.experimental.pallas import tpu_sc as plsc

T = 2048
D = 1024
F = 2048
E = 8
K = 2
S = T * K

BT = 256
NB = S // BT + E
NP = NB * BT

NC = 2
NS = 16
NW = NC * NS
RPW = NP // NW
TPW = T // NW


_RTB = 512


def _router_kernel(x_ref, wr_ref, eid_ref, w_ref):
    xb = x_ref[...]
    logits = jnp.dot(xb, wr_ref[...], preferred_element_type=jnp.float32)
    p = jax.nn.softmax(logits, axis=-1)
    ii = lax.broadcasted_iota(jnp.int32, p.shape, 1)
    m1 = jnp.max(p, axis=-1, keepdims=True)
    i1 = jnp.min(jnp.where(p >= m1, ii, E), axis=-1, keepdims=True)
    p2 = jnp.where(ii == i1, -jnp.inf, p)
    m2 = jnp.max(p2, axis=-1, keepdims=True)
    i2 = jnp.min(jnp.where(p2 >= m2, ii, E), axis=-1, keepdims=True)
    eid_ref[...] = jnp.concatenate([i1, i2], axis=1)
    w_ref[...] = jnp.concatenate([m1, m2], axis=1)


def _router(x, Wr):
    return pl.pallas_call(
        _router_kernel,
        grid=(T // _RTB,),
        in_specs=[
            pl.BlockSpec((_RTB, D), lambda t: (t, 0)),
            pl.BlockSpec((D, E), lambda t: (0, 0)),
        ],
        out_specs=[
            pl.BlockSpec((_RTB, K), lambda t: (t, 0)),
            pl.BlockSpec((_RTB, K), lambda t: (t, 0)),
        ],
        out_shape=[
            jax.ShapeDtypeStruct((T, K), jnp.int32),
            jax.ShapeDtypeStruct((T, K), jnp.float32),
        ],
    )(x, Wr)


def _lane(vec, i):
    lanes = lax.iota(jnp.int32, 16)
    return jnp.sum(jnp.where(lanes == i, vec, 0), axis=0)


PK = S + NP + 48
_GC = 48


def _route_gather_kernel(eidf_hbm, wf_hbm, x_hbm, pk_hbm, xs_hbm,
                         eid_v, wf_v, pk_v, tok_v, sh_tok, idx_v,
                         rows_a, rows_b, sga, sgb, soa, sob):
    cid = lax.axis_index("c")
    sid = lax.axis_index("s")
    lanes = lax.iota(jnp.int32, 16)

    @pl.when(sid == 0)
    def _routing():
        pltpu.sync_copy(eidf_hbm, eid_v)
        pltpu.sync_copy(wf_hbm, wf_v)

        def p1(i, c):
            v = eid_v[pl.ds(i * 16, 16)]
            for e in range(E):
                n = plsc.all_reduce_population_count(v == e)
                c = c + jnp.where(lanes == e, n, 0)
            return c

        cnt = lax.fori_loop(0, S // 16, p1, jnp.zeros((16,), jnp.int32))

        pcnt = ((cnt + (BT - 1)) // BT) * BT
        csum = plsc.cumsum(pcnt)
        padoff = csum - pcnt
        nu_blocks = _lane(csum, E - 1) // BT
        last_e = jnp.max(jnp.where(pcnt > 0, lanes, -1), axis=0)
        poff_s = [_lane(padoff, e) for e in range(E)]
        for r in range(NB // 16 + 1):
            bio = lanes + r * 16
            boff = bio * BT
            bev = jnp.zeros((16,), jnp.int32)
            for e in range(E):
                bev = bev + jnp.where(boff >= poff_s[e], 1, 0)
            bev = bev - 1
            bev = jnp.where(bio >= nu_blocks, last_e, bev)
            pk_v[pl.ds(S + NP + r * 16, 16)] = bev
        pk_v[pl.ds(S + NP + 32, 16)] = jnp.broadcast_to(nu_blocks, (16,))

        def pz(i, _):
            tok_v[pl.ds(i * 16, 16)] = jnp.zeros((16,), jnp.int32)
            return 0

        lax.fori_loop(0, NP // 16, pz, 0)

        def p2(i, oes):
            v = eid_v[pl.ds(i * 16, 16)]
            w = wf_v[pl.ds(i * 16, 16)]
            dest = jnp.zeros((16,), jnp.int32)
            nxt = []
            for e in range(E):
                m = v == e
                pc = plsc.cumsum(jnp.where(m, 1, 0))
                dest = jnp.where(m, oes[e] + pc - 1, dest)
                nxt.append(oes[e] + plsc.all_reduce_population_count(m))
            s_vec = lanes + i * 16
            loc = (s_vec & 1) * T + (s_vec >> 1)
            plsc.store_scatter(pk_v, [loc], dest)
            plsc.store_scatter(pk_v, [dest + S], plsc.bitcast(w, jnp.int32))
            plsc.store_scatter(tok_v, [dest], s_vec >> 1)
            return tuple(nxt)

        lax.fori_loop(0, S // 16, p2,
                      tuple(jnp.broadcast_to(poff_s[e], (16,))
                            for e in range(E)))

        @pl.when(cid == 0)
        def _publish_hbm():
            pltpu.sync_copy(pk_v, pk_hbm)

        pltpu.sync_copy(tok_v, sh_tok)

    plsc.subcore_barrier()

    wid = cid * NS + sid
    base = wid * RPW
    pltpu.sync_copy(sh_tok.at[pl.ds(base, RPW)], idx_v)

    def g(j, buf, s):
        return pltpu.async_copy(x_hbm.at[idx_v.at[pl.ds(j * _GC, _GC)]],
                                buf, s)

    def o(j, buf, s):
        return pltpu.async_copy(buf, xs_hbm.at[pl.ds(base + j * _GC, _GC)], s)

    ga = g(0, rows_a, sga)
    gb = g(1, rows_b, sgb)
    ga.wait()
    oa = o(0, rows_a, soa)
    gb.wait()
    ob = o(1, rows_b, sob)
    oa.wait()
    ga = g(2, rows_a, sga)
    ob.wait()
    gb = g(3, rows_b, sgb)
    ga.wait()
    oa = o(2, rows_a, soa)
    gb.wait()
    ob = o(3, rows_b, sob)
    oa.wait()
    ob.wait()


def _route_gather(eidf, wf, x):
    mesh = plsc.VectorSubcoreMesh(core_axis_name="c", subcore_axis_name="s")
    f = pl.kernel(
        _route_gather_kernel,
        out_type=[
            jax.ShapeDtypeStruct((PK,), jnp.int32),
            jax.ShapeDtypeStruct((NP, D), jnp.float32),
        ],
        mesh=mesh,
        scratch_types=[
            pltpu.VMEM((S,), jnp.int32),
            pltpu.VMEM((S,), jnp.float32),
            pltpu.VMEM((PK,), jnp.int32),
            pltpu.VMEM((NP,), jnp.int32),
            pltpu.VMEM_SHARED((NP,), jnp.int32),
            pltpu.VMEM((RPW,), jnp.int32),
            pltpu.VMEM((_GC, D), jnp.float32),
            pltpu.VMEM((_GC, D), jnp.float32),
            pltpu.SemaphoreType.DMA,
            pltpu.SemaphoreType.DMA,
            pltpu.SemaphoreType.DMA,
            pltpu.SemaphoreType.DMA,
        ],
        compiler_params=pltpu.CompilerParams(needs_layout_passes=False),
    )
    return f(eidf, wf, x)


def _ffn_kernel(be_ref, nu_ref, xs_ref, w1_ref, b1_ref, w2_ref, b2_ref,
                ws_ref, ys_ref):
    b = pl.program_id(0)

    @pl.when(b < nu_ref[0])
    def _():
        xb = xs_ref[...]
        h = jnp.dot(xb, w1_ref[0], preferred_element_type=jnp.float32)
        h = jax.nn.gelu(h + b1_ref[0])
        y = jnp.dot(h, w2_ref[0], preferred_element_type=jnp.float32)
        y = y + b2_ref[0]
        ys_ref[...] = y * ws_ref[0]


def _ffn(be, nu, xs, W1, b1r, W2, b2r, ws3):
    grid_spec = pltpu.PrefetchScalarGridSpec(
        num_scalar_prefetch=2,
        grid=(NB,),
        in_specs=[
            pl.BlockSpec((BT, D),
                         lambda b, be, nu: (jnp.minimum(b, nu[0] - 1), 0)),
            pl.BlockSpec((1, D, F), lambda b, be, nu: (be[b], 0, 0)),
            pl.BlockSpec((1, 1, F), lambda b, be, nu: (be[b], 0, 0)),
            pl.BlockSpec((1, F, D), lambda b, be, nu: (be[b], 0, 0)),
            pl.BlockSpec((1, 1, D), lambda b, be, nu: (be[b], 0, 0)),
            pl.BlockSpec((1, BT, 1), lambda b, be, nu: (b, 0, 0)),
        ],
        out_specs=pl.BlockSpec((BT, D), lambda b, be, nu: (b, 0)),
    )
    return pl.pallas_call(
        _ffn_kernel,
        grid_spec=grid_spec,
        out_shape=jax.ShapeDtypeStruct((NP, D), jnp.float32),
        compiler_params=pltpu.CompilerParams(
            dimension_semantics=("arbitrary",),
        ),
    )(be, nu, xs, W1, b1r, W2, b2r, ws3)


def _combine_kernel(ysw_hbm, destp_hbm, out_hbm, i0, i1,
                    a0, b0, a1, b1, sg0, sg1, so0, so1):
    cid = lax.axis_index("c")
    sid = lax.axis_index("s")
    wid = cid * NS + sid
    t0 = wid * TPW
    pltpu.sync_copy(destp_hbm.at[pl.ds(t0, TPW)], i0)
    pltpu.sync_copy(destp_hbm.at[pl.ds(T + t0, TPW)], i1)

    def gpair(ch, av, bv, s):
        ha = pltpu.async_copy(ysw_hbm.at[i0.at[pl.ds(ch * 16, 16)]], av, s)
        hb = pltpu.async_copy(ysw_hbm.at[i1.at[pl.ds(ch * 16, 16)]], bv, s)
        return ha, hb

    def addrows(av, bv):
        def add_row(j, _):
            for c in range(D // 16):
                sl = pl.ds(c * 16, 16)
                av[j, sl] = av[j, sl] + bv[j, sl]
            return 0

        lax.fori_loop(0, 16, add_row, 0)

    def out(ch, av, s):
        return pltpu.async_copy(av, out_hbm.at[pl.ds(t0 + ch * 16, 16)], s)

    h0 = gpair(0, a0, b0, sg0)
    h1 = gpair(1, a1, b1, sg1)
    h0[0].wait(); h0[1].wait()
    addrows(a0, b0)
    o0 = out(0, a0, so0)
    h1[0].wait(); h1[1].wait()
    addrows(a1, b1)
    o1 = out(1, a1, so1)
    o0.wait()
    h0 = gpair(2, a0, b0, sg0)
    o1.wait()
    h1 = gpair(3, a1, b1, sg1)
    h0[0].wait(); h0[1].wait()
    addrows(a0, b0)
    o0 = out(2, a0, so0)
    h1[0].wait(); h1[1].wait()
    addrows(a1, b1)
    o1 = out(3, a1, so1)
    o0.wait()
    o1.wait()


def _combine(ysw, destp):
    mesh = plsc.VectorSubcoreMesh(core_axis_name="c", subcore_axis_name="s")
    f = pl.kernel(
        _combine_kernel,
        out_type=jax.ShapeDtypeStruct((T, D), jnp.float32),
        mesh=mesh,
        scratch_types=[
            pltpu.VMEM((TPW,), jnp.int32),
            pltpu.VMEM((TPW,), jnp.int32),
            pltpu.VMEM((16, D), jnp.float32),
            pltpu.VMEM((16, D), jnp.float32),
            pltpu.VMEM((16, D), jnp.float32),
            pltpu.VMEM((16, D), jnp.float32),
            pltpu.SemaphoreType.DMA,
            pltpu.SemaphoreType.DMA,
            pltpu.SemaphoreType.DMA,
            pltpu.SemaphoreType.DMA,
        ],
        compiler_params=pltpu.CompilerParams(needs_layout_passes=False),
    )
    return f(ysw, destp)


def kernel(x, Wr, W1, b1, W2, b2):
    eid2, w2 = _router(x, Wr)
    eidf = eid2.reshape(S)
    wf = w2.reshape(S)
    pk, xs = _route_gather(eidf, wf, x)
    destp = pk[:S]
    ws = lax.bitcast_convert_type(pk[S:S + NP], jnp.float32)
    be = pk[S + NP:S + NP + 32]
    nu = pk[S + NP + 32:]
    b1r = b1.reshape(E, 1, F)
    b2r = b2.reshape(E, 1, D)
    ws3 = ws.reshape(NB, BT, 1)
    ysw = _ffn(be, nu, xs, W1, b1r, W2, b2r, ws3)
    return _combine(ysw, destp)
